# Initial kernel scaffold; baseline (speedup 1.0000x reference)
import jax
import jax.numpy as jnp
from jax.experimental import pallas as pl


def kernel(pf_points, pf_features, pf_mask, params):
    raise NotImplementedError("write your pallas kernel here")


# speedup vs baseline: 6.4715x; 6.4715x over previous
import functools

import jax
import jax.numpy as jnp
from jax import lax
from jax.experimental import pallas as pl
from jax.experimental.pallas import tpu as pltpu

P = 128
K = 7
BB = 4


def _topk_onehots(pd):
    col = lax.broadcasted_iota(jnp.int32, (P, P), 1)
    sels = []
    for t in range(K + 1):
        mx = jnp.max(pd, axis=1, keepdims=True)
        ismax = pd == mx
        idx = jnp.min(jnp.where(ismax, col, P), axis=1, keepdims=True)
        sel = col == idx
        if t > 0:
            sels.append(sel.astype(jnp.float32))
        pd = jnp.where(sel, -1e30, pd)
    return sels


def _edge_conv(pts, fts, layers, shortcut):
    xx = jnp.sum(pts * pts, axis=0, keepdims=True)
    inner = lax.dot_general(pts, pts, (((0,), (0,)), ((), ())),
                            preferred_element_type=jnp.float32)
    pd = 2.0 * inner - xx - xx.T

    sels = _topk_onehots(pd)
    gathered = [
        lax.dot_general(fts, s, (((1,), (1,)), ((), ())),
                        preferred_element_type=jnp.float32)
        for s in sels
    ]
    knn = jnp.concatenate(gathered, axis=1)
    xc = jnp.concatenate([fts] * K, axis=1)
    y = jnp.concatenate([xc, knn - xc], axis=0)
    for w, b in layers:
        y = jnp.maximum(
            lax.dot_general(w, y, (((1,), (0,)), ((), ())),
                            preferred_element_type=jnp.float32) + b, 0.0)
    acc = y[:, 0:P]
    for t in range(1, K):
        acc = acc + y[:, t * P:(t + 1) * P]
    mean = acc * (1.0 / K)
    if shortcut is None:
        scv = fts
    else:
        wsc, bsc = shortcut
        scv = lax.dot_general(wsc, fts, (((1,), (0,)), ((), ())),
                              preferred_element_type=jnp.float32) + bsc
    return jnp.maximum(scv + mean, 0.0)


def _net_body(pts_ref, f_ref, m_ref,
              a0_ref, c0_ref, sfts_ref, bfts_ref,
              e1w0_ref, e1b0_ref, e1w1_ref, e1b1_ref, e1w2_ref, e1b2_ref,
              e2w0_ref, e2b0_ref, e2w1_ref, e2b1_ref, e2w2_ref, e2b2_ref,
              scw_ref, scb_ref, fusw_ref, fusb_ref,
              fc1w_ref, fc1b_ref, fc2w_ref, fc2b_ref,
              out_ref):
    a0 = a0_ref[...]
    c0 = c0_ref[...]
    sfts = sfts_ref[...]
    bfts = bfts_ref[...]
    ec1 = [(e1w0_ref[...], e1b0_ref[...]),
           (e1w1_ref[...], e1b1_ref[...]),
           (e1w2_ref[...], e1b2_ref[...])]
    ec2 = [(e2w0_ref[...], e2b0_ref[...]),
           (e2w1_ref[...], e2b1_ref[...]),
           (e2w2_ref[...], e2b2_ref[...])]
    sc = (scw_ref[...], scb_ref[...])
    fusw = fusw_ref[...]
    fusb = fusb_ref[...]
    fc1w = fc1w_ref[...]
    fc1b = fc1b_ref[...]
    fc2w = fc2w_ref[...]
    fc2b = fc2b_ref[...]

    for j in range(BB):
        f = f_ref[j]
        m = m_ref[j]
        pts_in = pts_ref[j]

        x = f * m
        fts0 = jnp.maximum(
            lax.dot_general(a0, x, (((1,), (0,)), ((), ())),
                            preferred_element_type=jnp.float32) + c0, 0.0)
        features = fts0 * m
        points = pts_in * m
        coord_shift = jnp.where(m == 0.0, 1e9, 0.0)
        counts = jnp.maximum(jnp.sum(m, axis=1, keepdims=True), 1.0)
        fts = (features * sfts + bfts) * m

        out1 = _edge_conv(points + coord_shift, fts, ec1, None) * m
        out2 = _edge_conv(out1 + coord_shift, out1, ec2, sc) * m

        cat = jnp.concatenate([out1, out2], axis=0)
        ff = jnp.maximum(
            lax.dot_general(fusw, cat, (((1,), (0,)), ((), ())),
                            preferred_element_type=jnp.float32) + fusb,
            0.0) * m
        pooled = jnp.sum(ff, axis=1, keepdims=True) / counts
        h = jnp.maximum(
            lax.dot_general(fc1w, pooled, (((1,), (0,)), ((), ())),
                            preferred_element_type=jnp.float32) + fc1b, 0.0)
        o = lax.dot_general(fc2w, h, (((1,), (0,)), ((), ())),
                            preferred_element_type=jnp.float32) + fc2b
        out_ref[0, :, j:j + 1] = o


@jax.jit
def kernel(pf_points, pf_features, pf_mask, params):
    p = params
    B = pf_points.shape[0]
    inv = 1.0 / jnp.sqrt(1.0 + 1e-5)

    def fold(g, w):
        return (g * inv)[:, None] * w

    s0 = p['fcv_bn0_g'] * inv
    s1 = p['fcv_bn1_g'] * inv
    a0 = (s1[:, None] * p['fcv_w']) * s0[None, :]
    c0 = (s1 * (p['fcv_w'] @ p['fcv_bn0_b']) + p['fcv_bn1_b'])[:, None]
    sfts = (p['bn_fts_g'] * inv)[:, None]
    bfts = p['bn_fts_b'][:, None]
    e1w0 = fold(p['ec1_g0'], p['ec1_w0'])
    e1w1 = fold(p['ec1_g1'], p['ec1_w1'])
    e1w2 = fold(p['ec1_g2'], p['ec1_w2'])
    e2w0 = fold(p['ec2_g0'], p['ec2_w0'])
    e2w1 = fold(p['ec2_g1'], p['ec2_w1'])
    e2w2 = fold(p['ec2_g2'], p['ec2_w2'])
    scw = fold(p['ec2_scg'], p['ec2_scw'])
    fusw = fold(p['fus_g'], p['fus_w'])
    b = lambda v: v[:, None]

    ops = [
        a0, c0, sfts, bfts,
        e1w0, b(p['ec1_b0']), e1w1, b(p['ec1_b1']), e1w2, b(p['ec1_b2']),
        e2w0, b(p['ec2_b0']), e2w1, b(p['ec2_b1']), e2w2, b(p['ec2_b2']),
        scw, b(p['ec2_scb']), fusw, b(p['fus_b']),
        p['fc1_w'], b(p['fc1_b']), p['fc2_w'], b(p['fc2_b']),
    ]

    grid = (B // BB,)
    jet_spec = lambda c, d: pl.BlockSpec((BB, c, d), lambda i: (i, 0, 0))
    full = lambda arr: pl.BlockSpec(arr.shape, lambda i: (0,) * arr.ndim)

    out = pl.pallas_call(
        _net_body,
        grid=grid,
        in_specs=[jet_spec(2, P), jet_spec(5, P), jet_spec(1, P)]
        + [full(o) for o in ops],
        out_specs=pl.BlockSpec((1, 2, BB), lambda i: (i, 0, 0)),
        out_shape=jax.ShapeDtypeStruct((B // BB, 2, BB), jnp.float32),
        compiler_params=pltpu.CompilerParams(
            dimension_semantics=("arbitrary",)),
    )(pf_points, pf_features, pf_mask, *ops)
    return out.transpose(0, 2, 1).reshape(B, 2)


# speedup vs baseline: 21.5064x; 3.3232x over previous
import jax
import jax.numpy as jnp
from jax import lax
from jax.experimental import pallas as pl
from jax.experimental.pallas import tpu as pltpu

P = 128
K = 7
BB = 4


def _mm(a, b):
    return lax.dot_general(a, b, (((1,), (0,)), ((), ())),
                           preferred_element_type=jnp.float32)


def _topk_sel(pd):
    R = pd.shape[0]
    col = lax.broadcasted_iota(jnp.int32, (R, P), 1)
    sels = []
    for t in range(K + 1):
        mx = jnp.max(pd, axis=1, keepdims=True)
        ismax = pd == mx
        idx = jnp.min(jnp.where(ismax, col, P), axis=1, keepdims=True)
        sel = col == idx
        if t > 0:
            sels.append(sel.astype(jnp.float32))
        pd = jnp.where(sel, -1e30, pd)
    return sels


def _edge_conv(pts, fts, w0a_m_b, w0b, b0, layers, shortcut):
    C = fts.shape[0]
    xx = jnp.sum(pts * pts, axis=0, keepdims=True)
    pds = []
    for b in range(BB):
        pb = pts[:, b * P:(b + 1) * P]
        inner = lax.dot_general(pb, pb, (((0,), (0,)), ((), ())),
                                preferred_element_type=jnp.float32)
        pds.append(2.0 * inner - xx[:, b * P:(b + 1) * P])
    pd = jnp.concatenate(pds, axis=0)

    sels = _topk_sel(pd)
    knn_parts = []
    for b in range(BB):
        scat = jnp.concatenate([s[b * P:(b + 1) * P] for s in sels], axis=0)
        knn_parts.append(
            lax.dot_general(fts[:, b * P:(b + 1) * P], scat,
                            (((1,), (1,)), ((), ())),
                            preferred_element_type=jnp.float32))
    knn = jnp.concatenate(knn_parts, axis=1)

    v = _mm(w0a_m_b, fts)
    vtile = jnp.concatenate(
        [v[:, b * P:(b + 1) * P] for b in range(BB) for _ in range(K)],
        axis=1)
    y = jnp.maximum(_mm(w0b, knn) + vtile + b0, 0.0)
    for w, bb_ in layers:
        y = jnp.maximum(_mm(w, y) + bb_, 0.0)

    O = y.shape[0]
    mean_parts = []
    for b in range(BB):
        blk = y[:, b * K * P:(b + 1) * K * P]
        acc = blk[:, 0:P]
        for t in range(1, K):
            acc = acc + blk[:, t * P:(t + 1) * P]
        mean_parts.append(acc)
    mean = jnp.concatenate(mean_parts, axis=1) * (1.0 / K)

    if shortcut is None:
        scv = fts
    else:
        wsc, bsc = shortcut
        scv = _mm(wsc, fts) + bsc
    return jnp.maximum(scv + mean, 0.0)


def _net_body(pts_ref, f_ref, m_ref,
              a0_ref, c0_ref, sfts_ref, bfts_ref,
              e1w0a_ref, e1w0b_ref, e1b0_ref, e1w1_ref, e1b1_ref,
              e1w2_ref, e1b2_ref,
              e2w0a_ref, e2w0b_ref, e2b0_ref, e2w1_ref, e2b1_ref,
              e2w2_ref, e2b2_ref,
              scw_ref, scb_ref, fusw_ref, fusb_ref,
              fc1w_ref, fc1b_ref, fc2w_ref, fc2b_ref,
              out_ref):
    f = f_ref[...]
    m = m_ref[...]
    pts_in = pts_ref[...]

    x = f * m
    fts0 = jnp.maximum(_mm(a0_ref[...], x) + c0_ref[...], 0.0)
    features = fts0 * m
    points = pts_in * m
    coord_shift = jnp.where(m == 0.0, 1e9, 0.0)
    fts = (features * sfts_ref[...] + bfts_ref[...]) * m

    out1 = _edge_conv(
        points + coord_shift, fts,
        e1w0a_ref[...], e1w0b_ref[...], e1b0_ref[...],
        [(e1w1_ref[...], e1b1_ref[...]), (e1w2_ref[...], e1b2_ref[...])],
        None) * m
    out2 = _edge_conv(
        out1 + coord_shift, out1,
        e2w0a_ref[...], e2w0b_ref[...], e2b0_ref[...],
        [(e2w1_ref[...], e2b1_ref[...]), (e2w2_ref[...], e2b2_ref[...])],
        (scw_ref[...], scb_ref[...])) * m

    cat = jnp.concatenate([out1, out2], axis=0)
    ff = jnp.maximum(_mm(fusw_ref[...], cat) + fusb_ref[...], 0.0) * m

    pooled_parts = []
    cnt_parts = []
    for b in range(BB):
        pooled_parts.append(
            jnp.sum(ff[:, b * P:(b + 1) * P], axis=1, keepdims=True))
        cnt_parts.append(
            jnp.sum(m[:, b * P:(b + 1) * P], axis=1, keepdims=True))
    pooled = jnp.concatenate(pooled_parts, axis=1)
    counts = jnp.maximum(jnp.concatenate(cnt_parts, axis=1), 1.0)
    pooled = pooled / counts

    h = jnp.maximum(_mm(fc1w_ref[...], pooled) + fc1b_ref[...], 0.0)
    o = _mm(fc2w_ref[...], h) + fc2b_ref[...]
    out_ref[0] = o


@jax.jit
def kernel(pf_points, pf_features, pf_mask, params):
    p = params
    B = pf_points.shape[0]
    inv = 1.0 / jnp.sqrt(1.0 + 1e-5)

    def fold(g, w):
        return (g * inv)[:, None] * w

    s0 = p['fcv_bn0_g'] * inv
    s1 = p['fcv_bn1_g'] * inv
    a0 = (s1[:, None] * p['fcv_w']) * s0[None, :]
    c0 = (s1 * (p['fcv_w'] @ p['fcv_bn0_b']) + p['fcv_bn1_b'])[:, None]
    sfts = (p['bn_fts_g'] * inv)[:, None]
    bfts = p['bn_fts_b'][:, None]

    def split_w0(g, w0):
        wf = fold(g, w0)
        c = wf.shape[1] // 2
        w0a, w0b = wf[:, :c], wf[:, c:]
        return w0a - w0b, w0b

    e1w0a, e1w0b = split_w0(p['ec1_g0'], p['ec1_w0'])
    e2w0a, e2w0b = split_w0(p['ec2_g0'], p['ec2_w0'])
    e1w1 = fold(p['ec1_g1'], p['ec1_w1'])
    e1w2 = fold(p['ec1_g2'], p['ec1_w2'])
    e2w1 = fold(p['ec2_g1'], p['ec2_w1'])
    e2w2 = fold(p['ec2_g2'], p['ec2_w2'])
    scw = fold(p['ec2_scg'], p['ec2_scw'])
    fusw = fold(p['fus_g'], p['fus_w'])
    b = lambda v: v[:, None]

    ops = [
        a0, c0, sfts, bfts,
        e1w0a, e1w0b, b(p['ec1_b0']), e1w1, b(p['ec1_b1']),
        e1w2, b(p['ec1_b2']),
        e2w0a, e2w0b, b(p['ec2_b0']), e2w1, b(p['ec2_b1']),
        e2w2, b(p['ec2_b2']),
        scw, b(p['ec2_scb']), fusw, b(p['fus_b']),
        p['fc1_w'], b(p['fc1_b']), p['fc2_w'], b(p['fc2_b']),
    ]

    pts_f = pf_points.transpose(1, 0, 2).reshape(2, B * P)
    f_f = pf_features.transpose(1, 0, 2).reshape(5, B * P)
    m_f = pf_mask.transpose(1, 0, 2).reshape(1, B * P)

    grid = (B // BB,)
    col_spec = lambda c: pl.BlockSpec((c, BB * P), lambda i: (0, i))
    full = lambda arr: pl.BlockSpec(arr.shape, lambda i: (0,) * arr.ndim)

    out = pl.pallas_call(
        _net_body,
        grid=grid,
        in_specs=[col_spec(2), col_spec(5), col_spec(1)]
        + [full(o) for o in ops],
        out_specs=pl.BlockSpec((1, 2, BB), lambda i: (i, 0, 0)),
        out_shape=jax.ShapeDtypeStruct((B // BB, 2, BB), jnp.float32),
        compiler_params=pltpu.CompilerParams(
            dimension_semantics=("arbitrary",)),
    )(pts_f, f_f, m_f, *ops)
    return out.transpose(0, 2, 1).reshape(B, 2)


# speedup vs baseline: 26.6526x; 1.2393x over previous
import jax
import jax.numpy as jnp
from jax import lax
from jax.experimental import pallas as pl
from jax.experimental.pallas import tpu as pltpu

P = 128
K = 7
BB = 4


def _mm(a, b):
    return lax.dot_general(a, b, (((1,), (0,)), ((), ())),
                           preferred_element_type=jnp.float32)


def _topk_sel(pd):
    R = pd.shape[0]
    col = lax.broadcasted_iota(jnp.int32, (R, P), 1).astype(jnp.float32)
    sels = []
    for t in range(K + 1):
        mx = jnp.max(pd, axis=1, keepdims=True)
        cand = jnp.where(pd == mx, col, 1e9)
        idx = jnp.min(cand, axis=1, keepdims=True)
        sel = (col == idx).astype(jnp.float32)
        if t > 0:
            sels.append(sel)
        pd = pd - sel * 1e30
    return sels


def _edge_conv(pts, fts, w0a_m_b, w0b, b0, layers, shortcut):
    C = fts.shape[0]
    xx = jnp.sum(pts * pts, axis=0, keepdims=True)
    pds = []
    for b in range(BB):
        pb = pts[:, b * P:(b + 1) * P]
        inner = lax.dot_general(pb, pb, (((0,), (0,)), ((), ())),
                                preferred_element_type=jnp.float32)
        pds.append(2.0 * inner - xx[:, b * P:(b + 1) * P])
    pd = jnp.concatenate(pds, axis=0)

    sels = _topk_sel(pd)
    knn_parts = []
    for b in range(BB):
        scat = jnp.concatenate([s[b * P:(b + 1) * P] for s in sels], axis=0)
        knn_parts.append(
            lax.dot_general(fts[:, b * P:(b + 1) * P], scat,
                            (((1,), (1,)), ((), ())),
                            preferred_element_type=jnp.float32))
    knn = jnp.concatenate(knn_parts, axis=1)

    v = _mm(w0a_m_b, fts)
    vtile = jnp.concatenate(
        [v[:, b * P:(b + 1) * P] for b in range(BB) for _ in range(K)],
        axis=1)
    y = jnp.maximum(_mm(w0b, knn) + vtile + b0, 0.0)
    for w, bb_ in layers:
        y = jnp.maximum(_mm(w, y) + bb_, 0.0)

    O = y.shape[0]
    mean_parts = []
    for b in range(BB):
        blk = y[:, b * K * P:(b + 1) * K * P]
        acc = blk[:, 0:P]
        for t in range(1, K):
            acc = acc + blk[:, t * P:(t + 1) * P]
        mean_parts.append(acc)
    mean = jnp.concatenate(mean_parts, axis=1) * (1.0 / K)

    if shortcut is None:
        scv = fts
    else:
        wsc, bsc = shortcut
        scv = _mm(wsc, fts) + bsc
    return jnp.maximum(scv + mean, 0.0)


def _net_body(pts_ref, f_ref, m_ref,
              a0_ref, c0_ref, sfts_ref, bfts_ref,
              e1w0a_ref, e1w0b_ref, e1b0_ref, e1w1_ref, e1b1_ref,
              e1w2_ref, e1b2_ref,
              e2w0a_ref, e2w0b_ref, e2b0_ref, e2w1_ref, e2b1_ref,
              e2w2_ref, e2b2_ref,
              scw_ref, scb_ref, fusw_ref, fusb_ref,
              fc1w_ref, fc1b_ref, fc2w_ref, fc2b_ref,
              out_ref):
    f = f_ref[...]
    m = m_ref[...]
    pts_in = pts_ref[...]

    x = f * m
    fts0 = jnp.maximum(_mm(a0_ref[...], x) + c0_ref[...], 0.0)
    features = fts0 * m
    points = pts_in * m
    coord_shift = jnp.where(m == 0.0, 1e9, 0.0)
    fts = (features * sfts_ref[...] + bfts_ref[...]) * m

    out1 = _edge_conv(
        points + coord_shift, fts,
        e1w0a_ref[...], e1w0b_ref[...], e1b0_ref[...],
        [(e1w1_ref[...], e1b1_ref[...]), (e1w2_ref[...], e1b2_ref[...])],
        None) * m
    out2 = _edge_conv(
        out1 + coord_shift, out1,
        e2w0a_ref[...], e2w0b_ref[...], e2b0_ref[...],
        [(e2w1_ref[...], e2b1_ref[...]), (e2w2_ref[...], e2b2_ref[...])],
        (scw_ref[...], scb_ref[...])) * m

    cat = jnp.concatenate([out1, out2], axis=0)
    ff = jnp.maximum(_mm(fusw_ref[...], cat) + fusb_ref[...], 0.0) * m

    pooled_parts = []
    cnt_parts = []
    for b in range(BB):
        pooled_parts.append(
            jnp.sum(ff[:, b * P:(b + 1) * P], axis=1, keepdims=True))
        cnt_parts.append(
            jnp.sum(m[:, b * P:(b + 1) * P], axis=1, keepdims=True))
    pooled = jnp.concatenate(pooled_parts, axis=1)
    counts = jnp.maximum(jnp.concatenate(cnt_parts, axis=1), 1.0)
    pooled = pooled / counts

    h = jnp.maximum(_mm(fc1w_ref[...], pooled) + fc1b_ref[...], 0.0)
    o = _mm(fc2w_ref[...], h) + fc2b_ref[...]
    out_ref[0] = o


@jax.jit
def kernel(pf_points, pf_features, pf_mask, params):
    p = params
    B = pf_points.shape[0]
    inv = 1.0 / jnp.sqrt(1.0 + 1e-5)

    def fold(g, w):
        return (g * inv)[:, None] * w

    s0 = p['fcv_bn0_g'] * inv
    s1 = p['fcv_bn1_g'] * inv
    a0 = (s1[:, None] * p['fcv_w']) * s0[None, :]
    c0 = (s1 * (p['fcv_w'] @ p['fcv_bn0_b']) + p['fcv_bn1_b'])[:, None]
    sfts = (p['bn_fts_g'] * inv)[:, None]
    bfts = p['bn_fts_b'][:, None]

    def split_w0(g, w0):
        wf = fold(g, w0)
        c = wf.shape[1] // 2
        w0a, w0b = wf[:, :c], wf[:, c:]
        return w0a - w0b, w0b

    e1w0a, e1w0b = split_w0(p['ec1_g0'], p['ec1_w0'])
    e2w0a, e2w0b = split_w0(p['ec2_g0'], p['ec2_w0'])
    e1w1 = fold(p['ec1_g1'], p['ec1_w1'])
    e1w2 = fold(p['ec1_g2'], p['ec1_w2'])
    e2w1 = fold(p['ec2_g1'], p['ec2_w1'])
    e2w2 = fold(p['ec2_g2'], p['ec2_w2'])
    scw = fold(p['ec2_scg'], p['ec2_scw'])
    fusw = fold(p['fus_g'], p['fus_w'])
    b = lambda v: v[:, None]

    ops = [
        a0, c0, sfts, bfts,
        e1w0a, e1w0b, b(p['ec1_b0']), e1w1, b(p['ec1_b1']),
        e1w2, b(p['ec1_b2']),
        e2w0a, e2w0b, b(p['ec2_b0']), e2w1, b(p['ec2_b1']),
        e2w2, b(p['ec2_b2']),
        scw, b(p['ec2_scb']), fusw, b(p['fus_b']),
        p['fc1_w'], b(p['fc1_b']), p['fc2_w'], b(p['fc2_b']),
    ]

    pts_f = pf_points.transpose(1, 0, 2).reshape(2, B * P)
    f_f = pf_features.transpose(1, 0, 2).reshape(5, B * P)
    m_f = pf_mask.transpose(1, 0, 2).reshape(1, B * P)

    grid = (B // BB,)
    col_spec = lambda c: pl.BlockSpec((c, BB * P), lambda i: (0, i))
    full = lambda arr: pl.BlockSpec(arr.shape, lambda i: (0,) * arr.ndim)

    out = pl.pallas_call(
        _net_body,
        grid=grid,
        in_specs=[col_spec(2), col_spec(5), col_spec(1)]
        + [full(o) for o in ops],
        out_specs=pl.BlockSpec((1, 2, BB), lambda i: (i, 0, 0)),
        out_shape=jax.ShapeDtypeStruct((B // BB, 2, BB), jnp.float32),
        compiler_params=pltpu.CompilerParams(
            dimension_semantics=("arbitrary",)),
    )(pts_f, f_f, m_f, *ops)
    return out.transpose(0, 2, 1).reshape(B, 2)


# speedup vs baseline: 33.2909x; 1.2491x over previous
import jax
import jax.numpy as jnp
from jax import lax
from jax.experimental import pallas as pl
from jax.experimental.pallas import tpu as pltpu

P = 128
K = 7
BB = 8


def _mm(a, b):
    return lax.dot_general(a, b, (((1,), (0,)), ((), ())),
                           preferred_element_type=jnp.float32)


def _topk_sel(pd):
    R = pd.shape[0]
    col = lax.broadcasted_iota(jnp.int32, (R, P), 1).astype(jnp.float32)
    sels = []
    for t in range(K + 1):
        mx = jnp.max(pd, axis=1, keepdims=True)
        cand = jnp.where(pd == mx, col, 1e9)
        idx = jnp.min(cand, axis=1, keepdims=True)
        sel = (col == idx).astype(jnp.float32)
        if t > 0:
            sels.append(sel)
        pd = pd - sel * 1e30
    return sels


def _edge_conv(pts, fts, w0a_m_b, w0b, b0, layers, shortcut):
    C = fts.shape[0]
    xx = jnp.sum(pts * pts, axis=0, keepdims=True)
    pds = []
    for b in range(BB):
        pb = pts[:, b * P:(b + 1) * P]
        inner = lax.dot_general(pb, pb, (((0,), (0,)), ((), ())),
                                preferred_element_type=jnp.float32)
        pds.append(2.0 * inner - xx[:, b * P:(b + 1) * P])
    pd = jnp.concatenate(pds, axis=0)

    sels = _topk_sel(pd)
    knn_parts = []
    for b in range(BB):
        scat = jnp.concatenate([s[b * P:(b + 1) * P] for s in sels], axis=0)
        knn_parts.append(
            lax.dot_general(fts[:, b * P:(b + 1) * P], scat,
                            (((1,), (1,)), ((), ())),
                            preferred_element_type=jnp.float32))
    knn = jnp.concatenate(knn_parts, axis=1)

    v = _mm(w0a_m_b, fts)
    vtile = jnp.concatenate(
        [v[:, b * P:(b + 1) * P] for b in range(BB) for _ in range(K)],
        axis=1)
    y = jnp.maximum(_mm(w0b, knn) + vtile + b0, 0.0)
    for w, bb_ in layers:
        y = jnp.maximum(_mm(w, y) + bb_, 0.0)

    O = y.shape[0]
    mean_parts = []
    for b in range(BB):
        blk = y[:, b * K * P:(b + 1) * K * P]
        acc = blk[:, 0:P]
        for t in range(1, K):
            acc = acc + blk[:, t * P:(t + 1) * P]
        mean_parts.append(acc)
    mean = jnp.concatenate(mean_parts, axis=1) * (1.0 / K)

    if shortcut is None:
        scv = fts
    else:
        wsc, bsc = shortcut
        scv = _mm(wsc, fts) + bsc
    return jnp.maximum(scv + mean, 0.0)


def _net_body(pts_ref, f_ref, m_ref,
              a0_ref, c0_ref, sfts_ref, bfts_ref,
              e1w0a_ref, e1w0b_ref, e1b0_ref, e1w1_ref, e1b1_ref,
              e1w2_ref, e1b2_ref,
              e2w0a_ref, e2w0b_ref, e2b0_ref, e2w1_ref, e2b1_ref,
              e2w2_ref, e2b2_ref,
              scw_ref, scb_ref, fusw_ref, fusb_ref,
              fc1w_ref, fc1b_ref, fc2w_ref, fc2b_ref,
              out_ref):
    f = f_ref[...]
    m = m_ref[...]
    pts_in = pts_ref[...]

    x = f * m
    fts0 = jnp.maximum(_mm(a0_ref[...], x) + c0_ref[...], 0.0)
    features = fts0 * m
    points = pts_in * m
    coord_shift = jnp.where(m == 0.0, 1e9, 0.0)
    fts = (features * sfts_ref[...] + bfts_ref[...]) * m

    out1 = _edge_conv(
        points + coord_shift, fts,
        e1w0a_ref[...], e1w0b_ref[...], e1b0_ref[...],
        [(e1w1_ref[...], e1b1_ref[...]), (e1w2_ref[...], e1b2_ref[...])],
        None) * m
    out2 = _edge_conv(
        out1 + coord_shift, out1,
        e2w0a_ref[...], e2w0b_ref[...], e2b0_ref[...],
        [(e2w1_ref[...], e2b1_ref[...]), (e2w2_ref[...], e2b2_ref[...])],
        (scw_ref[...], scb_ref[...])) * m

    cat = jnp.concatenate([out1, out2], axis=0)
    ff = jnp.maximum(_mm(fusw_ref[...], cat) + fusb_ref[...], 0.0) * m

    pooled_parts = []
    cnt_parts = []
    for b in range(BB):
        pooled_parts.append(
            jnp.sum(ff[:, b * P:(b + 1) * P], axis=1, keepdims=True))
        cnt_parts.append(
            jnp.sum(m[:, b * P:(b + 1) * P], axis=1, keepdims=True))
    pooled = jnp.concatenate(pooled_parts, axis=1)
    counts = jnp.maximum(jnp.concatenate(cnt_parts, axis=1), 1.0)
    pooled = pooled / counts

    h = jnp.maximum(_mm(fc1w_ref[...], pooled) + fc1b_ref[...], 0.0)
    o = _mm(fc2w_ref[...], h) + fc2b_ref[...]
    out_ref[0] = o


@jax.jit
def kernel(pf_points, pf_features, pf_mask, params):
    p = params
    B = pf_points.shape[0]
    inv = 1.0 / jnp.sqrt(1.0 + 1e-5)

    def fold(g, w):
        return (g * inv)[:, None] * w

    s0 = p['fcv_bn0_g'] * inv
    s1 = p['fcv_bn1_g'] * inv
    a0 = (s1[:, None] * p['fcv_w']) * s0[None, :]
    c0 = (s1 * (p['fcv_w'] @ p['fcv_bn0_b']) + p['fcv_bn1_b'])[:, None]
    sfts = (p['bn_fts_g'] * inv)[:, None]
    bfts = p['bn_fts_b'][:, None]

    def split_w0(g, w0):
        wf = fold(g, w0)
        c = wf.shape[1] // 2
        w0a, w0b = wf[:, :c], wf[:, c:]
        return w0a - w0b, w0b

    e1w0a, e1w0b = split_w0(p['ec1_g0'], p['ec1_w0'])
    e2w0a, e2w0b = split_w0(p['ec2_g0'], p['ec2_w0'])
    e1w1 = fold(p['ec1_g1'], p['ec1_w1'])
    e1w2 = fold(p['ec1_g2'], p['ec1_w2'])
    e2w1 = fold(p['ec2_g1'], p['ec2_w1'])
    e2w2 = fold(p['ec2_g2'], p['ec2_w2'])
    scw = fold(p['ec2_scg'], p['ec2_scw'])
    fusw = fold(p['fus_g'], p['fus_w'])
    b = lambda v: v[:, None]

    ops = [
        a0, c0, sfts, bfts,
        e1w0a, e1w0b, b(p['ec1_b0']), e1w1, b(p['ec1_b1']),
        e1w2, b(p['ec1_b2']),
        e2w0a, e2w0b, b(p['ec2_b0']), e2w1, b(p['ec2_b1']),
        e2w2, b(p['ec2_b2']),
        scw, b(p['ec2_scb']), fusw, b(p['fus_b']),
        p['fc1_w'], b(p['fc1_b']), p['fc2_w'], b(p['fc2_b']),
    ]

    pts_f = pf_points.transpose(1, 0, 2).reshape(2, B * P)
    f_f = pf_features.transpose(1, 0, 2).reshape(5, B * P)
    m_f = pf_mask.transpose(1, 0, 2).reshape(1, B * P)

    grid = (B // BB,)
    col_spec = lambda c: pl.BlockSpec((c, BB * P), lambda i: (0, i))
    full = lambda arr: pl.BlockSpec(arr.shape, lambda i: (0,) * arr.ndim)

    out = pl.pallas_call(
        _net_body,
        grid=grid,
        in_specs=[col_spec(2), col_spec(5), col_spec(1)]
        + [full(o) for o in ops],
        out_specs=pl.BlockSpec((1, 2, BB), lambda i: (i, 0, 0)),
        out_shape=jax.ShapeDtypeStruct((B // BB, 2, BB), jnp.float32),
        compiler_params=pltpu.CompilerParams(
            dimension_semantics=("arbitrary",)),
    )(pts_f, f_f, m_f, *ops)
    return out.transpose(0, 2, 1).reshape(B, 2)


# speedup vs baseline: 37.7527x; 1.1340x over previous
import jax
import jax.numpy as jnp
from jax import lax
from jax.experimental import pallas as pl
from jax.experimental.pallas import tpu as pltpu

P = 128
K = 7
BB = 16


def _mm(a, b):
    return lax.dot_general(a, b, (((1,), (0,)), ((), ())),
                           preferred_element_type=jnp.float32)


def _topk_sel(pd):
    R = pd.shape[0]
    col = lax.broadcasted_iota(jnp.int32, (R, P), 1).astype(jnp.float32)
    sels = []
    for t in range(K + 1):
        mx = jnp.max(pd, axis=1, keepdims=True)
        cand = jnp.where(pd == mx, col, 1e9)
        idx = jnp.min(cand, axis=1, keepdims=True)
        sel = (col == idx).astype(jnp.float32)
        if t > 0:
            sels.append(sel)
        pd = pd - sel * 1e30
    return sels


def _edge_conv(pts, fts, w0a_m_b, w0b, b0, layers, shortcut):
    C = fts.shape[0]
    xx = jnp.sum(pts * pts, axis=0, keepdims=True)
    pds = []
    for b in range(BB):
        pb = pts[:, b * P:(b + 1) * P]
        inner = lax.dot_general(pb, pb, (((0,), (0,)), ((), ())),
                                preferred_element_type=jnp.float32)
        pds.append(2.0 * inner - xx[:, b * P:(b + 1) * P])
    pd = jnp.concatenate(pds, axis=0)

    sels = _topk_sel(pd)
    knn_parts = []
    for b in range(BB):
        scat = jnp.concatenate([s[b * P:(b + 1) * P] for s in sels], axis=0)
        knn_parts.append(
            lax.dot_general(fts[:, b * P:(b + 1) * P], scat,
                            (((1,), (1,)), ((), ())),
                            preferred_element_type=jnp.float32))
    knn = jnp.concatenate(knn_parts, axis=1)

    v = _mm(w0a_m_b, fts)
    vtile = jnp.concatenate(
        [v[:, b * P:(b + 1) * P] for b in range(BB) for _ in range(K)],
        axis=1)
    y = jnp.maximum(_mm(w0b, knn) + vtile + b0, 0.0)
    for w, bb_ in layers:
        y = jnp.maximum(_mm(w, y) + bb_, 0.0)

    O = y.shape[0]
    mean_parts = []
    for b in range(BB):
        blk = y[:, b * K * P:(b + 1) * K * P]
        acc = blk[:, 0:P]
        for t in range(1, K):
            acc = acc + blk[:, t * P:(t + 1) * P]
        mean_parts.append(acc)
    mean = jnp.concatenate(mean_parts, axis=1) * (1.0 / K)

    if shortcut is None:
        scv = fts
    else:
        wsc, bsc = shortcut
        scv = _mm(wsc, fts) + bsc
    return jnp.maximum(scv + mean, 0.0)


def _net_body(pts_ref, f_ref, m_ref,
              a0_ref, c0_ref, sfts_ref, bfts_ref,
              e1w0a_ref, e1w0b_ref, e1b0_ref, e1w1_ref, e1b1_ref,
              e1w2_ref, e1b2_ref,
              e2w0a_ref, e2w0b_ref, e2b0_ref, e2w1_ref, e2b1_ref,
              e2w2_ref, e2b2_ref,
              scw_ref, scb_ref, fusw_ref, fusb_ref,
              fc1w_ref, fc1b_ref, fc2w_ref, fc2b_ref,
              out_ref):
    f = f_ref[...]
    m = m_ref[...]
    pts_in = pts_ref[...]

    x = f * m
    fts0 = jnp.maximum(_mm(a0_ref[...], x) + c0_ref[...], 0.0)
    features = fts0 * m
    points = pts_in * m
    coord_shift = jnp.where(m == 0.0, 1e9, 0.0)
    fts = (features * sfts_ref[...] + bfts_ref[...]) * m

    out1 = _edge_conv(
        points + coord_shift, fts,
        e1w0a_ref[...], e1w0b_ref[...], e1b0_ref[...],
        [(e1w1_ref[...], e1b1_ref[...]), (e1w2_ref[...], e1b2_ref[...])],
        None) * m
    out2 = _edge_conv(
        out1 + coord_shift, out1,
        e2w0a_ref[...], e2w0b_ref[...], e2b0_ref[...],
        [(e2w1_ref[...], e2b1_ref[...]), (e2w2_ref[...], e2b2_ref[...])],
        (scw_ref[...], scb_ref[...])) * m

    cat = jnp.concatenate([out1, out2], axis=0)
    ff = jnp.maximum(_mm(fusw_ref[...], cat) + fusb_ref[...], 0.0) * m

    pooled_parts = []
    cnt_parts = []
    for b in range(BB):
        pooled_parts.append(
            jnp.sum(ff[:, b * P:(b + 1) * P], axis=1, keepdims=True))
        cnt_parts.append(
            jnp.sum(m[:, b * P:(b + 1) * P], axis=1, keepdims=True))
    pooled = jnp.concatenate(pooled_parts, axis=1)
    counts = jnp.maximum(jnp.concatenate(cnt_parts, axis=1), 1.0)
    pooled = pooled / counts

    h = jnp.maximum(_mm(fc1w_ref[...], pooled) + fc1b_ref[...], 0.0)
    o = _mm(fc2w_ref[...], h) + fc2b_ref[...]
    out_ref[0] = o


@jax.jit
def kernel(pf_points, pf_features, pf_mask, params):
    p = params
    B = pf_points.shape[0]
    inv = 1.0 / jnp.sqrt(1.0 + 1e-5)

    def fold(g, w):
        return (g * inv)[:, None] * w

    s0 = p['fcv_bn0_g'] * inv
    s1 = p['fcv_bn1_g'] * inv
    a0 = (s1[:, None] * p['fcv_w']) * s0[None, :]
    c0 = (s1 * (p['fcv_w'] @ p['fcv_bn0_b']) + p['fcv_bn1_b'])[:, None]
    sfts = (p['bn_fts_g'] * inv)[:, None]
    bfts = p['bn_fts_b'][:, None]

    def split_w0(g, w0):
        wf = fold(g, w0)
        c = wf.shape[1] // 2
        w0a, w0b = wf[:, :c], wf[:, c:]
        return w0a - w0b, w0b

    e1w0a, e1w0b = split_w0(p['ec1_g0'], p['ec1_w0'])
    e2w0a, e2w0b = split_w0(p['ec2_g0'], p['ec2_w0'])
    e1w1 = fold(p['ec1_g1'], p['ec1_w1'])
    e1w2 = fold(p['ec1_g2'], p['ec1_w2'])
    e2w1 = fold(p['ec2_g1'], p['ec2_w1'])
    e2w2 = fold(p['ec2_g2'], p['ec2_w2'])
    scw = fold(p['ec2_scg'], p['ec2_scw'])
    fusw = fold(p['fus_g'], p['fus_w'])
    b = lambda v: v[:, None]

    ops = [
        a0, c0, sfts, bfts,
        e1w0a, e1w0b, b(p['ec1_b0']), e1w1, b(p['ec1_b1']),
        e1w2, b(p['ec1_b2']),
        e2w0a, e2w0b, b(p['ec2_b0']), e2w1, b(p['ec2_b1']),
        e2w2, b(p['ec2_b2']),
        scw, b(p['ec2_scb']), fusw, b(p['fus_b']),
        p['fc1_w'], b(p['fc1_b']), p['fc2_w'], b(p['fc2_b']),
    ]

    pts_f = pf_points.transpose(1, 0, 2).reshape(2, B * P)
    f_f = pf_features.transpose(1, 0, 2).reshape(5, B * P)
    m_f = pf_mask.transpose(1, 0, 2).reshape(1, B * P)

    grid = (B // BB,)
    col_spec = lambda c: pl.BlockSpec((c, BB * P), lambda i: (0, i))
    full = lambda arr: pl.BlockSpec(arr.shape, lambda i: (0,) * arr.ndim)

    out = pl.pallas_call(
        _net_body,
        grid=grid,
        in_specs=[col_spec(2), col_spec(5), col_spec(1)]
        + [full(o) for o in ops],
        out_specs=pl.BlockSpec((1, 2, BB), lambda i: (i, 0, 0)),
        out_shape=jax.ShapeDtypeStruct((B // BB, 2, BB), jnp.float32),
        compiler_params=pltpu.CompilerParams(
            dimension_semantics=("arbitrary",)),
    )(pts_f, f_f, m_f, *ops)
    return out.transpose(0, 2, 1).reshape(B, 2)


# speedup vs baseline: 40.5345x; 1.0737x over previous
import jax
import jax.numpy as jnp
from jax import lax
from jax.experimental import pallas as pl
from jax.experimental.pallas import tpu as pltpu

P = 128
K = 7
BB = 32


def _mm(a, b):
    return lax.dot_general(a, b, (((1,), (0,)), ((), ())),
                           preferred_element_type=jnp.float32)


def _topk_sel(pd):
    R = pd.shape[0]
    col = lax.broadcasted_iota(jnp.int32, (R, P), 1).astype(jnp.float32)
    sels = []
    for t in range(K + 1):
        mx = jnp.max(pd, axis=1, keepdims=True)
        cand = jnp.where(pd == mx, col, 1e9)
        idx = jnp.min(cand, axis=1, keepdims=True)
        sel = (col == idx).astype(jnp.float32)
        if t > 0:
            sels.append(sel)
        pd = pd - sel * 1e30
    return sels


def _edge_conv(pts, fts, w0a_m_b, w0b, b0, layers, shortcut):
    C = fts.shape[0]
    xx = jnp.sum(pts * pts, axis=0, keepdims=True)
    pds = []
    for b in range(BB):
        pb = pts[:, b * P:(b + 1) * P]
        inner = lax.dot_general(pb, pb, (((0,), (0,)), ((), ())),
                                preferred_element_type=jnp.float32)
        pds.append(2.0 * inner - xx[:, b * P:(b + 1) * P])
    pd = jnp.concatenate(pds, axis=0)

    sels = _topk_sel(pd)
    knn_parts = []
    for b in range(BB):
        scat = jnp.concatenate([s[b * P:(b + 1) * P] for s in sels], axis=0)
        knn_parts.append(
            lax.dot_general(fts[:, b * P:(b + 1) * P], scat,
                            (((1,), (1,)), ((), ())),
                            preferred_element_type=jnp.float32))
    knn = jnp.concatenate(knn_parts, axis=1)

    v = _mm(w0a_m_b, fts)
    vtile = jnp.concatenate(
        [v[:, b * P:(b + 1) * P] for b in range(BB) for _ in range(K)],
        axis=1)
    y = jnp.maximum(_mm(w0b, knn) + vtile + b0, 0.0)
    for w, bb_ in layers:
        y = jnp.maximum(_mm(w, y) + bb_, 0.0)

    O = y.shape[0]
    mean_parts = []
    for b in range(BB):
        blk = y[:, b * K * P:(b + 1) * K * P]
        acc = blk[:, 0:P]
        for t in range(1, K):
            acc = acc + blk[:, t * P:(t + 1) * P]
        mean_parts.append(acc)
    mean = jnp.concatenate(mean_parts, axis=1) * (1.0 / K)

    if shortcut is None:
        scv = fts
    else:
        wsc, bsc = shortcut
        scv = _mm(wsc, fts) + bsc
    return jnp.maximum(scv + mean, 0.0)


def _net_body(pts_ref, f_ref, m_ref,
              a0_ref, c0_ref, sfts_ref, bfts_ref,
              e1w0a_ref, e1w0b_ref, e1b0_ref, e1w1_ref, e1b1_ref,
              e1w2_ref, e1b2_ref,
              e2w0a_ref, e2w0b_ref, e2b0_ref, e2w1_ref, e2b1_ref,
              e2w2_ref, e2b2_ref,
              scw_ref, scb_ref, fusw_ref, fusb_ref,
              fc1w_ref, fc1b_ref, fc2w_ref, fc2b_ref,
              out_ref):
    f = f_ref[...]
    m = m_ref[...]
    pts_in = pts_ref[...]

    x = f * m
    fts0 = jnp.maximum(_mm(a0_ref[...], x) + c0_ref[...], 0.0)
    features = fts0 * m
    points = pts_in * m
    coord_shift = jnp.where(m == 0.0, 1e9, 0.0)
    fts = (features * sfts_ref[...] + bfts_ref[...]) * m

    out1 = _edge_conv(
        points + coord_shift, fts,
        e1w0a_ref[...], e1w0b_ref[...], e1b0_ref[...],
        [(e1w1_ref[...], e1b1_ref[...]), (e1w2_ref[...], e1b2_ref[...])],
        None) * m
    out2 = _edge_conv(
        out1 + coord_shift, out1,
        e2w0a_ref[...], e2w0b_ref[...], e2b0_ref[...],
        [(e2w1_ref[...], e2b1_ref[...]), (e2w2_ref[...], e2b2_ref[...])],
        (scw_ref[...], scb_ref[...])) * m

    cat = jnp.concatenate([out1, out2], axis=0)
    ff = jnp.maximum(_mm(fusw_ref[...], cat) + fusb_ref[...], 0.0) * m

    pooled_parts = []
    cnt_parts = []
    for b in range(BB):
        pooled_parts.append(
            jnp.sum(ff[:, b * P:(b + 1) * P], axis=1, keepdims=True))
        cnt_parts.append(
            jnp.sum(m[:, b * P:(b + 1) * P], axis=1, keepdims=True))
    pooled = jnp.concatenate(pooled_parts, axis=1)
    counts = jnp.maximum(jnp.concatenate(cnt_parts, axis=1), 1.0)
    pooled = pooled / counts

    h = jnp.maximum(_mm(fc1w_ref[...], pooled) + fc1b_ref[...], 0.0)
    o = _mm(fc2w_ref[...], h) + fc2b_ref[...]
    out_ref[0] = o


@jax.jit
def kernel(pf_points, pf_features, pf_mask, params):
    p = params
    B = pf_points.shape[0]
    inv = 1.0 / jnp.sqrt(1.0 + 1e-5)

    def fold(g, w):
        return (g * inv)[:, None] * w

    s0 = p['fcv_bn0_g'] * inv
    s1 = p['fcv_bn1_g'] * inv
    a0 = (s1[:, None] * p['fcv_w']) * s0[None, :]
    c0 = (s1 * (p['fcv_w'] @ p['fcv_bn0_b']) + p['fcv_bn1_b'])[:, None]
    sfts = (p['bn_fts_g'] * inv)[:, None]
    bfts = p['bn_fts_b'][:, None]

    def split_w0(g, w0):
        wf = fold(g, w0)
        c = wf.shape[1] // 2
        w0a, w0b = wf[:, :c], wf[:, c:]
        return w0a - w0b, w0b

    e1w0a, e1w0b = split_w0(p['ec1_g0'], p['ec1_w0'])
    e2w0a, e2w0b = split_w0(p['ec2_g0'], p['ec2_w0'])
    e1w1 = fold(p['ec1_g1'], p['ec1_w1'])
    e1w2 = fold(p['ec1_g2'], p['ec1_w2'])
    e2w1 = fold(p['ec2_g1'], p['ec2_w1'])
    e2w2 = fold(p['ec2_g2'], p['ec2_w2'])
    scw = fold(p['ec2_scg'], p['ec2_scw'])
    fusw = fold(p['fus_g'], p['fus_w'])
    b = lambda v: v[:, None]

    ops = [
        a0, c0, sfts, bfts,
        e1w0a, e1w0b, b(p['ec1_b0']), e1w1, b(p['ec1_b1']),
        e1w2, b(p['ec1_b2']),
        e2w0a, e2w0b, b(p['ec2_b0']), e2w1, b(p['ec2_b1']),
        e2w2, b(p['ec2_b2']),
        scw, b(p['ec2_scb']), fusw, b(p['fus_b']),
        p['fc1_w'], b(p['fc1_b']), p['fc2_w'], b(p['fc2_b']),
    ]

    pts_f = pf_points.transpose(1, 0, 2).reshape(2, B * P)
    f_f = pf_features.transpose(1, 0, 2).reshape(5, B * P)
    m_f = pf_mask.transpose(1, 0, 2).reshape(1, B * P)

    grid = (B // BB,)
    col_spec = lambda c: pl.BlockSpec((c, BB * P), lambda i: (0, i))
    full = lambda arr: pl.BlockSpec(arr.shape, lambda i: (0,) * arr.ndim)

    out = pl.pallas_call(
        _net_body,
        grid=grid,
        in_specs=[col_spec(2), col_spec(5), col_spec(1)]
        + [full(o) for o in ops],
        out_specs=pl.BlockSpec((1, 2, BB), lambda i: (i, 0, 0)),
        out_shape=jax.ShapeDtypeStruct((B // BB, 2, BB), jnp.float32),
        compiler_params=pltpu.CompilerParams(
            dimension_semantics=("arbitrary",)),
    )(pts_f, f_f, m_f, *ops)
    return out.transpose(0, 2, 1).reshape(B, 2)
